# baseline (speedup 1.0000x reference)
import functools

import jax
import jax.numpy as jnp
from jax import lax
from jax.experimental import pallas as pl
from jax.experimental.pallas import tpu as pltpu
from jax.experimental.pallas import tpu_sc as plsc

N = 10000
NP = 10240
E = 320000
F_IN = 128
HID = 64
A_OUT = 8

NC = 2
NS = 16
NW = NC * NS
CH = 125
NBUF = 8
NCHUNK = 80
PER_TILE = CH * NCHUNK
RPT = NP // NS
NGRP = NCHUNK // NBUF

_mesh = plsc.VectorSubcoreMesh(core_axis_name="c", subcore_axis_name="s")
_sc_params = pltpu.CompilerParams(use_tc_tiling_on_sc=False)
_sc_params_nl = pltpu.CompilerParams(use_tc_tiling_on_sc=False,
                                     needs_layout_passes=False)


DCHUNK = E // (NS * CH)


@functools.partial(
    pl.kernel,
    mesh=_mesh,
    out_type=[
        jax.ShapeDtypeStruct((NC, NP, HID), jnp.bfloat16),
        jax.ShapeDtypeStruct((NC, NS, RPT), jnp.float32),
    ],
    compiler_params=_sc_params_nl,
    scratch_types=(
        [pltpu.VMEM((NCHUNK, CH), jnp.int32)] * 2
        + [pltpu.VMEM((128,), jnp.float32)]
        + [pltpu.VMEM((CH, HID), jnp.bfloat16)] * NBUF
        + [pltpu.VMEM((RPT, HID), jnp.bfloat16)]
        + [pltpu.VMEM((RPT,), jnp.float32)] * 2
        + [pltpu.VMEM_SHARED((NP,), jnp.float32)]
        + [pltpu.VMEM_SHARED((NP, HID), jnp.bfloat16)] * 2
        + [pltpu.SemaphoreType.DMA] * (1 + 2 * NBUF)
    ),
)
def _deg_agg_kernel(dstd_hbm, src_hbm, dst_hbm, hb_hbm, zeros_deg, zeros_mat,
                    out_hbm, dis_hbm, *refs):
    src_v, dst_v = refs[0], refs[1]
    ones_v = refs[2]
    rows = refs[3:3 + NBUF]
    scalebuf = refs[3 + NBUF]
    degbuf, disbuf = refs[4 + NBUF], refs[5 + NBUF]
    acc_deg = refs[6 + NBUF]
    gsh, acc = refs[7 + NBUF], refs[8 + NBUF]
    dsem = refs[9 + NBUF]
    gsem = refs[10 + NBUF:10 + 2 * NBUF]
    ssem = refs[10 + 2 * NBUF:10 + 3 * NBUF]
    cid = lax.axis_index("c")
    sid = lax.axis_index("s")
    wid = sid * NC + cid

    for i in range(128 // 16):
        ones_v[pl.ds(i * 16, 16)] = jnp.full((16,), 1.0, jnp.float32)
    pltpu.sync_copy(zeros_deg, acc_deg.at[pl.ds(sid * RPT, RPT)])
    pltpu.sync_copy(zeros_mat, acc.at[pl.ds(sid * RPT, RPT)])
    @pl.when(sid < NS - 1)
    def _():
        pltpu.sync_copy(hb_hbm.at[pl.ds(sid * RPT, RPT)],
                        gsh.at[pl.ds(sid * RPT, RPT)])

    @pl.when(sid == NS - 1)
    def _():
        pltpu.sync_copy(hb_hbm.at[pl.ds((NS - 1) * RPT, N - (NS - 1) * RPT)],
                        gsh.at[pl.ds((NS - 1) * RPT, N - (NS - 1) * RPT)])

    plsc.subcore_barrier()

    def dbody(c, carry):
        pltpu.async_copy(ones_v.at[pl.ds(0, CH)], acc_deg.at[dst_v.at[c]],
                         dsem, add=True)

        @pl.when(c >= NBUF)
        def _():
            pltpu.make_async_copy(ones_v.at[pl.ds(0, CH)],
                                  acc_deg.at[dst_v.at[0]], dsem).wait()

        return carry

    for r in range(2):
        pltpu.sync_copy(dstd_hbm.at[sid, r], dst_v)
        lax.fori_loop(0, NCHUNK, dbody, 0)
        for _ in range(NBUF):
            pltpu.make_async_copy(ones_v.at[pl.ds(0, CH)],
                                  acc_deg.at[dst_v.at[0]], dsem).wait()
    pltpu.sync_copy(src_hbm.at[wid], src_v)
    pltpu.sync_copy(dst_hbm.at[wid], dst_v)
    plsc.subcore_barrier()

    pltpu.sync_copy(acc_deg.at[pl.ds(sid * RPT, RPT)], degbuf)
    for k in range(RPT // 16):
        d = degbuf[pl.ds(k * 16, 16)] + 1.0
        i32 = plsc.bitcast(d, jnp.int32)
        y = plsc.bitcast(0x5F3759DF - lax.shift_right_arithmetic(i32, 1),
                         jnp.float32)
        y = y * (1.5 - 0.5 * d * y * y)
        y = y * (1.5 - 0.5 * d * y * y)
        disbuf[pl.ds(k * 16, 16)] = y
    pltpu.sync_copy(disbuf, dis_hbm.at[cid, sid])
    pltpu.sync_copy(gsh.at[pl.ds(sid * RPT, RPT)], scalebuf)

    def sblk(k, carry):
        dv = disbuf[pl.ds(k * 16, 16)]
        for j in range(16):
            sv = jnp.full((16,), dv[j], jnp.float32)
            s = plsc.pack(sv, sv, format=plsc.PackFormat.INTERLEAVED)
            i = k * 16 + j
            v0 = scalebuf[i, pl.ds(0, 32)]
            scalebuf[i, pl.ds(0, 32)] = v0 * s
            v1 = scalebuf[i, pl.ds(32, 32)]
            scalebuf[i, pl.ds(32, 32)] = v1 * s
        return carry

    lax.fori_loop(0, RPT // 16, sblk, 0)
    pltpu.sync_copy(scalebuf, gsh.at[pl.ds(sid * RPT, RPT)])
    plsc.subcore_barrier()

    for b in range(NBUF):
        pltpu.async_copy(gsh.at[src_v.at[b]], rows[b], gsem[b])

    def group(gi, carry):
        c0 = gi * NBUF
        for b in range(NBUF):
            pltpu.make_async_copy(gsh.at[src_v.at[c0 + b]], rows[b],
                                  gsem[b]).wait()
            pltpu.async_copy(rows[b], acc.at[dst_v.at[c0 + b]], ssem[b],
                             add=True)
        for b in range(NBUF):
            pltpu.make_async_copy(rows[b], acc.at[dst_v.at[c0 + b]],
                                  ssem[b]).wait()
            pltpu.async_copy(gsh.at[src_v.at[c0 + NBUF + b]], rows[b],
                             gsem[b])
        return carry

    lax.fori_loop(0, NGRP - 1, group, 0)

    c0 = NCHUNK - NBUF
    for b in range(NBUF):
        pltpu.make_async_copy(gsh.at[src_v.at[c0 + b]], rows[b],
                              gsem[b]).wait()
        pltpu.async_copy(rows[b], acc.at[dst_v.at[c0 + b]], ssem[b], add=True)
    for b in range(NBUF):
        pltpu.make_async_copy(rows[b], acc.at[dst_v.at[c0 + b]],
                              ssem[b]).wait()

    plsc.subcore_barrier()
    pltpu.sync_copy(acc.at[pl.ds(sid * RPT, RPT)],
                    out_hbm.at[cid, pl.ds(sid * RPT, RPT)])


@functools.partial(
    pl.kernel,
    mesh=_mesh,
    out_type=jax.ShapeDtypeStruct((NC, NP, HID), jnp.bfloat16),
    compiler_params=_sc_params,
    scratch_types=(
        [pltpu.VMEM((NCHUNK, CH), jnp.int32)] * 2
        + [pltpu.VMEM((CH, HID), jnp.bfloat16)] * NBUF
        + [pltpu.VMEM_SHARED((N, HID), jnp.bfloat16)]
        + [pltpu.VMEM_SHARED((NP, HID), jnp.bfloat16)]
        + [pltpu.SemaphoreType.DMA] * (2 * NBUF)
    ),
)
def _agg_kernel(src_hbm, dst_hbm, g_hbm, zeros_hbm, out_hbm, *refs):
    src_v, dst_v = refs[0], refs[1]
    rows = refs[2:2 + NBUF]
    gsh = refs[2 + NBUF]
    acc = refs[3 + NBUF]
    gsem = refs[4 + NBUF:4 + 2 * NBUF]
    ssem = refs[4 + 2 * NBUF:4 + 3 * NBUF]
    cid = lax.axis_index("c")
    sid = lax.axis_index("s")
    wid = sid * NC + cid

    pltpu.sync_copy(zeros_hbm, acc.at[pl.ds(sid * RPT, RPT)])
    pltpu.sync_copy(g_hbm.at[pl.ds(sid * (N // NS), N // NS)],
                    gsh.at[pl.ds(sid * (N // NS), N // NS)])
    pltpu.sync_copy(src_hbm.at[wid], src_v)
    pltpu.sync_copy(dst_hbm.at[wid], dst_v)
    plsc.subcore_barrier()

    for b in range(NBUF):
        pltpu.async_copy(gsh.at[src_v.at[b]], rows[b], gsem[b])

    def group(gi, carry):
        c0 = gi * NBUF
        for b in range(NBUF):
            pltpu.make_async_copy(gsh.at[src_v.at[c0 + b]], rows[b],
                                  gsem[b]).wait()
            pltpu.async_copy(rows[b], acc.at[dst_v.at[c0 + b]], ssem[b],
                             add=True)
        for b in range(NBUF):
            pltpu.make_async_copy(rows[b], acc.at[dst_v.at[c0 + b]],
                                  ssem[b]).wait()
            pltpu.async_copy(gsh.at[src_v.at[c0 + NBUF + b]], rows[b],
                             gsem[b])
        return carry

    lax.fori_loop(0, NGRP - 1, group, 0)

    c0 = NCHUNK - NBUF
    for b in range(NBUF):
        pltpu.make_async_copy(gsh.at[src_v.at[c0 + b]], rows[b],
                              gsem[b]).wait()
        pltpu.async_copy(rows[b], acc.at[dst_v.at[c0 + b]], ssem[b], add=True)
    for b in range(NBUF):
        pltpu.make_async_copy(rows[b], acc.at[dst_v.at[c0 + b]],
                              ssem[b]).wait()

    plsc.subcore_barrier()
    pltpu.sync_copy(acc.at[pl.ds(sid * RPT, RPT)],
                    out_hbm.at[cid, pl.ds(sid * RPT, RPT)])


R = 1000


def _mm1_body(x, w1, h_out, hb_out):
    h = jnp.dot(x[...], w1[...], preferred_element_type=jnp.float32)
    h_out[...] = h
    hb_out[...] = h.astype(jnp.bfloat16)


def _mm1(x, w1):
    return pl.pallas_call(
        _mm1_body,
        grid=(N // R,),
        in_specs=[
            pl.BlockSpec((R, F_IN), lambda i: (i, 0)),
            pl.BlockSpec((F_IN, HID), lambda i: (0, 0)),
        ],
        out_specs=[
            pl.BlockSpec((R, HID), lambda i: (i, 0)),
            pl.BlockSpec((R, HID), lambda i: (i, 0)),
        ],
        out_shape=[
            jax.ShapeDtypeStruct((N, HID), jnp.float32),
            jax.ShapeDtypeStruct((N, HID), jnp.bfloat16),
        ],
    )(x, w1)


def _dense2_body(agga, aggb, h, dis, b, w, out, gb_out):
    agg = agga[...].astype(jnp.float32) + aggb[...].astype(jnp.float32)
    g1 = h[...] * dis[...]
    t = jnp.maximum(dis[...] * (agg + g1) + b[...], 0.0)
    g2 = jnp.dot(t, w[...], preferred_element_type=jnp.float32) * dis[...]
    out[...] = g2
    gb_out[...] = g2.astype(jnp.bfloat16)


def _dense2(agga, aggb, g, dis, b, w):
    return pl.pallas_call(
        _dense2_body,
        grid=(N // R,),
        in_specs=[
            pl.BlockSpec((R, HID), lambda i: (i, 0)),
            pl.BlockSpec((R, HID), lambda i: (i, 0)),
            pl.BlockSpec((R, HID), lambda i: (i, 0)),
            pl.BlockSpec((R, 1), lambda i: (i, 0)),
            pl.BlockSpec((1, HID), lambda i: (0, 0)),
            pl.BlockSpec((HID, HID), lambda i: (0, 0)),
        ],
        out_specs=[
            pl.BlockSpec((R, HID), lambda i: (i, 0)),
            pl.BlockSpec((R, HID), lambda i: (i, 0)),
        ],
        out_shape=[
            jax.ShapeDtypeStruct((N, HID), jnp.float32),
            jax.ShapeDtypeStruct((N, HID), jnp.bfloat16),
        ],
    )(agga, aggb, g, dis, b, w)


def _dense3_body(agga, aggb, g, dis, b, w, bh, out):
    agg = agga[...].astype(jnp.float32) + aggb[...].astype(jnp.float32)
    t = jnp.maximum(dis[...] * (agg + g[...]) + b[...], 0.0)
    out[...] = jnp.dot(t, w[...], preferred_element_type=jnp.float32) + bh[...]


def _dense3(agga, aggb, g, dis, b, w, bh):
    return pl.pallas_call(
        _dense3_body,
        grid=(N // R,),
        in_specs=[
            pl.BlockSpec((R, HID), lambda i: (i, 0)),
            pl.BlockSpec((R, HID), lambda i: (i, 0)),
            pl.BlockSpec((R, HID), lambda i: (i, 0)),
            pl.BlockSpec((R, 1), lambda i: (i, 0)),
            pl.BlockSpec((1, HID), lambda i: (0, 0)),
            pl.BlockSpec((HID, A_OUT), lambda i: (0, 0)),
            pl.BlockSpec((1, A_OUT), lambda i: (0, 0)),
        ],
        out_specs=pl.BlockSpec((R, A_OUT), lambda i: (i, 0)),
        out_shape=jax.ShapeDtypeStruct((N, A_OUT), jnp.float32),
    )(agga, aggb, g, dis, b, w, bh)


def kernel(x, edge_index, W1, b1, W2, b2, Wh, bh):
    src3 = edge_index[0].reshape(NW, NCHUNK, CH)
    dst3 = edge_index[1].reshape(NW, NCHUNK, CH)
    dstd = edge_index[1].reshape(NS, 2, NCHUNK, CH)
    zeros_row = jnp.zeros((RPT,), jnp.float32)
    zeros_mat = jnp.zeros((RPT, HID), jnp.bfloat16)

    h1, h1b = _mm1(x, W1)
    agg1, diso = _deg_agg_kernel(dstd, src3, dst3, h1b, zeros_row, zeros_mat)
    dis = diso[0].reshape(NP)[:N].reshape(N, 1)
    g2, g2b = _dense2(agg1[0, :N], agg1[1, :N], h1, dis,
                      b1.reshape(1, HID), W2)

    agg2 = _agg_kernel(src3, dst3, g2b, zeros_mat)
    return _dense3(agg2[0, :N], agg2[1, :N], g2, dis,
                   b2.reshape(1, HID), Wh, bh.reshape(1, A_OUT))
